# baseline (speedup 1.0000x reference)
import functools

import jax
import jax.numpy as jnp
from jax import lax
from jax.experimental import pallas as pl
from jax.experimental.pallas import tpu as pltpu
from jax.experimental.pallas import tpu_sc as plsc

_INFO = plsc.get_sparse_core_info()
_NC, _NS = _INFO.num_cores, _INFO.num_subcores
_NW = _NC * _NS

_IDX_W = 128
_K = 8


@functools.partial(jax.jit, static_argnums=(2, 3))
def _gather_rows(table, idx2d, rows, d):
    rows_per_w = rows // _NW
    groups = rows_per_w // _K
    mesh = plsc.VectorSubcoreMesh(core_axis_name="c", subcore_axis_name="s")

    @functools.partial(
        pl.kernel,
        mesh=mesh,
        out_type=jax.ShapeDtypeStruct((rows, _IDX_W, d), jnp.float32),
        scratch_types=[
            pltpu.VMEM((_K, _IDX_W), jnp.int32),
            pltpu.VMEM((_K, _IDX_W, d), jnp.float32),
            pltpu.SemaphoreType.DMA,
        ],
        compiler_params=pltpu.CompilerParams(use_tc_tiling_on_sc=False),
    )
    def k(table_hbm, idx_hbm, out_hbm, idx_v, rows_v, sem):
        wid = lax.axis_index("s") * _NC + lax.axis_index("c")
        base = wid * rows_per_w

        def group(g, carry):
            row0 = base + g * _K
            pltpu.sync_copy(idx_hbm.at[pl.ds(row0, _K)], idx_v)
            cps = [
                pltpu.async_copy(table_hbm.at[idx_v.at[j]], rows_v.at[j], sem)
                for j in range(_K)
            ]
            for c in cps:
                c.wait()
            pltpu.sync_copy(rows_v, out_hbm.at[pl.ds(row0, _K)])
            return carry

        lax.fori_loop(0, groups, group, 0)

    return k(table, idx2d)


def kernel(x, table):
    b, s = x.shape
    v, d = table.shape
    n = b * s
    idx2d = x.reshape(n // _IDX_W, _IDX_W).astype(jnp.int32)
    out = _gather_rows(table, idx2d, n // _IDX_W, d)
    return out.reshape(b, s, d)
